# speedup vs baseline: 6.9246x; 6.9246x over previous
import jax
import jax.numpy as jnp
from jax.experimental import pallas as pl


def _copy_body(x_ref, o_ref):
    o_ref[...] = x_ref[...]


def kernel(x):
    n, c = x.shape
    br = 8192
    nb = n // br
    return pl.pallas_call(
        _copy_body,
        grid=(nb,),
        in_specs=[pl.BlockSpec((br, c), lambda i: ((i + nb // 2) % nb, 0))],
        out_specs=pl.BlockSpec((br, c), lambda i: (i, 0)),
        out_shape=jax.ShapeDtypeStruct(x.shape, x.dtype),
    )(x)
